# baseline (speedup 1.0000x reference)
import functools

import jax
import jax.numpy as jnp
from jax import lax
from jax.experimental import pallas as pl
from jax.experimental.pallas import tpu as pltpu
from jax.experimental.pallas import tpu_sc as plsc

NC = 2
NS = 16
NW = NC * NS

B = 4096 * 200
D = 64

CHUNK = 128
K = 8
STEP = CHUNK * K
B_PER_W = B // NW
N_STEPS = B_PER_W // STEP


def kernel(x, table):
    idx = x.reshape(B // CHUNK, CHUNK).astype(jnp.int32)
    mesh = plsc.VectorSubcoreMesh(core_axis_name="c", subcore_axis_name="s")

    @functools.partial(
        pl.kernel,
        mesh=mesh,
        out_type=jax.ShapeDtypeStruct((B, D), jnp.float32),
        scratch_types=[
            pltpu.VMEM((K, CHUNK), jnp.int32),
            pltpu.VMEM((STEP, D), jnp.float32),
            pltpu.SemaphoreType.DMA,
        ],
        compiler_params=pltpu.CompilerParams(use_tc_tiling_on_sc=False),
    )
    def gather_kernel(idx_hbm, table_hbm, out_hbm, idx_v, rows_v, gsem):
        wid = lax.axis_index("s") * NC + lax.axis_index("c")
        row_base = wid * (N_STEPS * K)
        out_base = wid * B_PER_W

        @pl.loop(0, N_STEPS)
        def _(i):
            pltpu.sync_copy(idx_hbm.at[pl.ds(row_base + i * K, K)], idx_v)
            handles = []
            for j in range(K):
                handles.append(
                    pltpu.async_copy(
                        table_hbm.at[idx_v.at[j]],
                        rows_v.at[pl.ds(j * CHUNK, CHUNK)],
                        gsem,
                    )
                )
            for h in handles:
                h.wait()
            pltpu.sync_copy(rows_v, out_hbm.at[pl.ds(out_base + i * STEP, STEP)])

    out = gather_kernel(idx, table)
    return out.reshape(x.shape[0], x.shape[1], D)


# speedup vs baseline: 1.0099x; 1.0099x over previous
import functools

import jax
import jax.numpy as jnp
from jax import lax
from jax.experimental import pallas as pl
from jax.experimental.pallas import tpu as pltpu
from jax.experimental.pallas import tpu_sc as plsc

NC = 2
NS = 16
NW = NC * NS

B = 4096 * 200
D = 64

CHUNK = 128
K = 2
NBUF = 5
STEP = CHUNK * K
SUPER = NBUF * STEP
B_PER_W = B // NW
N_SUPER = B_PER_W // SUPER
assert B_PER_W % SUPER == 0


def kernel(x, table):
    idx = x.reshape(B // CHUNK, CHUNK).astype(jnp.int32)
    mesh = plsc.VectorSubcoreMesh(core_axis_name="c", subcore_axis_name="s")

    @functools.partial(
        pl.kernel,
        mesh=mesh,
        out_type=jax.ShapeDtypeStruct((B, D), jnp.float32),
        scratch_types=[
            pltpu.VMEM((NBUF * K, CHUNK), jnp.int32),
            pltpu.VMEM((NBUF, STEP, D), jnp.float32),
        ]
        + [pltpu.SemaphoreType.DMA] * (2 * NBUF),
        compiler_params=pltpu.CompilerParams(use_tc_tiling_on_sc=False),
    )
    def gather_kernel(idx_hbm, table_hbm, out_hbm, idx_v, rows_v, *sems):
        gsems = sems[:NBUF]
        osems = sems[NBUF:]
        wid = lax.axis_index("s") * NC + lax.axis_index("c")
        idx_row_base = wid * (N_SUPER * NBUF * K)
        out_base = wid * B_PER_W

        @pl.loop(0, N_SUPER)
        def _(t):
            pltpu.sync_copy(
                idx_hbm.at[pl.ds(idx_row_base + t * (NBUF * K), NBUF * K)],
                idx_v,
            )
            handles = []
            for b in range(NBUF):
                @pl.when(t > 0)
                def _(b=b):
                    pltpu.make_async_copy(
                        rows_v.at[b],
                        out_hbm.at[pl.ds(out_base, STEP)],
                        osems[b],
                    ).wait()

                for j in range(K):
                    handles.append(
                        pltpu.async_copy(
                            table_hbm.at[idx_v.at[b * K + j]],
                            rows_v.at[b].at[pl.ds(j * CHUNK, CHUNK)],
                            gsems[b],
                        )
                    )
            for b in range(NBUF):
                for j in range(K):
                    handles[b * K + j].wait()
                pltpu.async_copy(
                    rows_v.at[b],
                    out_hbm.at[pl.ds(out_base + (t * NBUF + b) * STEP, STEP)],
                    osems[b],
                )

        for b in range(NBUF):
            pltpu.make_async_copy(
                rows_v.at[b],
                out_hbm.at[pl.ds(out_base, STEP)],
                osems[b],
            ).wait()

    out = gather_kernel(idx, table)
    return out.reshape(x.shape[0], x.shape[1], D)


# speedup vs baseline: 1.2281x; 1.2160x over previous
import functools

import jax
import jax.numpy as jnp
from jax import lax
from jax.experimental import pallas as pl
from jax.experimental.pallas import tpu as pltpu
from jax.experimental.pallas import tpu_sc as plsc

NC = 2
NS = 16
NW = NC * NS

B = 4096 * 200
D = 64
DW = 128

CHUNK = 128
NBUF = 5
B_PER_W = B // NW
SUPER = NBUF * CHUNK
N_SUPER = B_PER_W // SUPER
assert B_PER_W % SUPER == 0


def kernel(x, table):
    idx = x.reshape(B // CHUNK, CHUNK).astype(jnp.int32)
    tablew = jnp.pad(table, ((0, 0), (0, DW - D)))
    mesh = plsc.VectorSubcoreMesh(core_axis_name="c", subcore_axis_name="s")

    @functools.partial(
        pl.kernel,
        mesh=mesh,
        out_type=jax.ShapeDtypeStruct((B, DW), jnp.float32),
        scratch_types=[
            pltpu.VMEM((NBUF, CHUNK), jnp.int32),
            pltpu.VMEM((NBUF, CHUNK, DW), jnp.float32),
        ]
        + [pltpu.SemaphoreType.DMA] * (2 * NBUF),
        compiler_params=pltpu.CompilerParams(use_tc_tiling_on_sc=False),
    )
    def gather_kernel(idx_hbm, table_hbm, out_hbm, idx_v, rows_v, *sems):
        gsems = sems[:NBUF]
        osems = sems[NBUF:]
        wid = lax.axis_index("s") * NC + lax.axis_index("c")
        idx_row_base = wid * (N_SUPER * NBUF)
        out_base = wid * B_PER_W

        @pl.loop(0, N_SUPER)
        def _(t):
            pltpu.sync_copy(
                idx_hbm.at[pl.ds(idx_row_base + t * NBUF, NBUF)],
                idx_v,
            )
            handles = []
            for b in range(NBUF):
                @pl.when(t > 0)
                def _(b=b):
                    pltpu.make_async_copy(
                        rows_v.at[b],
                        out_hbm.at[pl.ds(out_base, CHUNK)],
                        osems[b],
                    ).wait()

                handles.append(
                    pltpu.async_copy(
                        table_hbm.at[idx_v.at[b]],
                        rows_v.at[b],
                        gsems[b],
                    )
                )
            for b in range(NBUF):
                handles[b].wait()
                pltpu.async_copy(
                    rows_v.at[b],
                    out_hbm.at[pl.ds(out_base + (t * NBUF + b) * CHUNK, CHUNK)],
                    osems[b],
                )

        for b in range(NBUF):
            pltpu.make_async_copy(
                rows_v.at[b],
                out_hbm.at[pl.ds(out_base, CHUNK)],
                osems[b],
            ).wait()

    outw = gather_kernel(idx, tablew)
    return outw[:, :D].reshape(x.shape[0], x.shape[1], D)
